# baseline (speedup 1.0000x reference)
import functools

import numpy as np

import jax
import jax.numpy as jnp
from jax import lax
from jax.experimental import pallas as pl
from jax.experimental.pallas import tpu as pltpu
from jax.experimental.pallas import tpu_sc as plsc

_NOISE_STD = 1.0
_P_MEAN = -1.2
_P_STD = 1.5
_SIGMA_DATA = 0.5

_B = 64
_L = 1024
_NC = 2
_NS = 16
_ROWS_PER_W = _B // (_NC * _NS)
_NT = _L // 16


def _sc_body(x_hbm, basep_hbm, ns_hbm, mask_hbm, lcn_hbm, stab_hbm,
             xn_hbm, nz_hbm, sig_hbm, wt_hbm,
             v_x0, v_x1, v_bp0, v_bp1, v_nz0, v_nz1, v_xn0, v_xn1,
             v_mask0, v_mask1, v_ns0, v_ns1, v_sig0, v_sig1,
             v_wt0, v_wt1, v_lcn0, v_lcn1, v_stab,
             sem_in0, sem_in1, sem_out):
    wid = lax.axis_index("s") * _NC + lax.axis_index("c")
    pltpu.sync_copy(stab_hbm, v_stab)

    iota = lax.iota(jnp.int32, 16)
    i3 = iota * 3

    bufs = (
        (v_x0, v_bp0, v_nz0, v_xn0, v_mask0, v_ns0, v_sig0, v_wt0, v_lcn0),
        (v_x1, v_bp1, v_nz1, v_xn1, v_mask1, v_ns1, v_sig1, v_wt1, v_lcn1),
    )
    sems_in = (sem_in0, sem_in1)
    in_handles = []
    for rr in range(_ROWS_PER_W):
        r = wid * _ROWS_PER_W + rr
        sem = sems_in[rr]
        b_x, b_bp, b_nz, b_xn, b_mask, b_ns, b_sig, b_wt, b_lcn = bufs[rr]
        in_handles.append([
            pltpu.async_copy(x_hbm.at[r], b_x, sem),
            pltpu.async_copy(basep_hbm.at[r], b_bp, sem),
            pltpu.async_copy(ns_hbm.at[r], b_ns, sem),
            pltpu.async_copy(mask_hbm.at[r], b_mask, sem),
            pltpu.async_copy(lcn_hbm.at[r], b_lcn, sem),
        ])

    out_handles = []
    for rr in range(_ROWS_PER_W):
        r = wid * _ROWS_PER_W + rr
        for h in in_handles[rr]:
            h.wait()
        b_x, b_bp, b_nz, b_xn, b_mask, b_ns, b_sig, b_wt, b_lcn = bufs[rr]
        stab_v = plsc.load_gather(v_stab, [jnp.full((16,), r, jnp.int32)])

        z = jnp.zeros((16,), jnp.float32)

        @plsc.parallel_loop(0, _NT, unroll=4, carry=(z, z, z, z))
        def pass_a(t, carry):
            accn, ax, ay, az = carry
            o = t * 16
            k = 1.0 - b_mask[pl.ds(o, 16)]
            gx = b_bp[pl.ds(o, 16)]
            gy = b_bp[pl.ds(_L + o, 16)]
            gz = b_bp[pl.ds(2 * _L + o, 16)]
            return (accn + k, ax + k * gx, ay + k * gy, az + k * gz)

        accn, ax, ay, az = pass_a
        n_at = jnp.sum(accn)
        inv = (1.0 - stab_v) / jnp.maximum(jnp.broadcast_to(n_at, (16,)), 1.0)
        cx = jnp.broadcast_to(jnp.sum(ax), (16,)) * inv
        cy = jnp.broadcast_to(jnp.sum(ay), (16,)) * inv
        cz = jnp.broadcast_to(jnp.sum(az), (16,)) * inv

        @plsc.parallel_loop(0, _NT, unroll=4)
        def pass_b(t):
            o = t * 16
            k = 1.0 - b_mask[pl.ds(o, 16)]
            s = _SIGMA_DATA * jnp.exp(b_ns[pl.ds(o, 16)] * _P_STD + _P_MEAN)
            b_sig[pl.ds(o, 16)] = s
            b_wt[pl.ds(o, 16)] = 4.0 + 1.0 / (s * s)
            off = t * 48
            for c, cc in enumerate((cx, cy, cz)):
                idx = i3 + (off + c)
                nf = k * (b_bp[pl.ds(c * _L + o, 16)] - cc)
                plsc.store_scatter(b_nz, [idx], nf)
                xg = plsc.load_gather(b_x, [idx])
                plsc.store_scatter(b_xn, [idx], xg + s * nf)

        @pl.when(stab_v[0] > 0.5)
        def _():
            na_i = n_at.astype(jnp.int32)
            for chunk in range(2):
                i = iota + chunk * 16
                j = i // 3
                c = i - j * 3
                b2 = ((j >> 2) & 1).astype(jnp.float32) - 0.5
                b1 = ((j >> 1) & 1).astype(jnp.float32) - 0.5
                b0 = (j & 1).astype(jnp.float32) - 0.5
                g0 = plsc.load_gather(b_lcn, [c])
                d4 = plsc.load_gather(b_lcn, [c + 12]) - g0
                d2 = plsc.load_gather(b_lcn, [c + 6]) - g0
                d1 = plsc.load_gather(b_lcn, [c + 3]) - g0
                corner = b2 * d4 + b1 * d2 + b0 * d1
                lidx = jnp.minimum(na_i + j, _L - 1)
                tidx = lidx * 3 + c
                valid = i < 24
                plsc.store_scatter(b_nz, [tidx], corner, mask=valid)
                sg = plsc.load_gather(b_sig, [lidx])
                xg = plsc.load_gather(b_x, [tidx])
                plsc.store_scatter(b_xn, [tidx], xg + sg * corner, mask=valid)

        out_handles.extend([
            pltpu.async_copy(b_nz, nz_hbm.at[r], sem_out),
            pltpu.async_copy(b_xn, xn_hbm.at[r], sem_out),
            pltpu.async_copy(b_sig, sig_hbm.at[r], sem_out),
            pltpu.async_copy(b_wt, wt_hbm.at[r], sem_out),
        ])

    for h in out_handles:
        h.wait()


_f32 = jnp.float32
_R = _ROWS_PER_W
_sc_call = pl.kernel(
    _sc_body,
    out_type=(
        jax.ShapeDtypeStruct((_B, 3 * _L), _f32),
        jax.ShapeDtypeStruct((_B, 3 * _L), _f32),
        jax.ShapeDtypeStruct((_B, _L), _f32),
        jax.ShapeDtypeStruct((_B, _L), _f32),
    ),
    mesh=plsc.VectorSubcoreMesh(
        core_axis_name="c", subcore_axis_name="s",
        num_cores=_NC, num_subcores=_NS),
    compiler_params=pltpu.CompilerParams(
        needs_layout_passes=False, disable_bounds_checks=True,
        use_tc_tiling_on_sc=True),
    scratch_types=(
        [pltpu.VMEM((3 * _L,), _f32)] * 8
        + [pltpu.VMEM((_L,), _f32)] * 8
        + [pltpu.VMEM((32,), _f32)] * 2
        + [pltpu.VMEM((_B,), _f32)]
        + [pltpu.SemaphoreType.DMA] * 3
    ),
)


def _np_threefry2x32(k0, k1, x0, x1):
    k0 = np.uint32(k0); k1 = np.uint32(k1)
    ks = (k0, k1, np.uint32(k0 ^ k1 ^ np.uint32(0x1BD11BDA)))
    x0 = (x0 + ks[0]).astype(np.uint32)
    x1 = (x1 + ks[1]).astype(np.uint32)
    rots = ([13, 15, 26, 6], [17, 29, 16, 24])
    for i in range(5):
        for r in rots[i % 2]:
            x0 = (x0 + x1).astype(np.uint32)
            x1 = ((x1 << np.uint32(r)) | (x1 >> np.uint32(32 - r))).astype(np.uint32)
            x1 = (x1 ^ x0).astype(np.uint32)
        x0 = (x0 + ks[(i + 1) % 3]).astype(np.uint32)
        x1 = (x1 + ks[(i + 2) % 3] + np.uint32(i + 1)).astype(np.uint32)
    return x0, x1


def _np_random_bits(k0, k1, n):
    o0, o1 = _np_threefry2x32(
        k0, k1, np.zeros(n, np.uint32), np.arange(n, dtype=np.uint32))
    return (o0 ^ o1).astype(np.uint32)


def _np_erfinv_f32(x):
    x = x.astype(np.float32)
    w = (-np.log((np.float32(1.0) - x) * (np.float32(1.0) + x))).astype(np.float32)
    wa = (w - np.float32(2.5)).astype(np.float32)
    p = np.float32(2.81022636e-08)
    for c in (3.43273939e-07, -3.5233877e-06, -4.39150654e-06, 0.00021858087,
              -0.00125372503, -0.00417768164, 0.246640727, 1.50140941):
        p = (np.float32(c) + p * wa).astype(np.float32)
    pa = p
    wb = (np.sqrt(w) - np.float32(3.0)).astype(np.float32)
    p = np.float32(-0.000200214257)
    for c in (0.000100950558, 0.00134934322, -0.00367342844, 0.00573950773,
              -0.0076224613, 0.00943887047, 1.00167406, 2.83297682):
        p = (np.float32(c) + p * wb).astype(np.float32)
    pb = p
    return (np.where(w < np.float32(5.0), pa, pb) * x).astype(np.float32)


def _np_normal(k0, k1, shape):
    n = int(np.prod(shape))
    bits = _np_random_bits(k0, k1, n)
    fb = ((bits >> np.uint32(9)) | np.uint32(0x3F800000)).view(np.float32)
    floats = (fb - np.float32(1.0)).astype(np.float32)
    lo = np.float32(np.nextafter(np.float32(-1.0), np.float32(0.0)))
    u = np.maximum(lo, (floats * (np.float32(1.0) - lo) + lo).astype(np.float32))
    return (np.float32(np.sqrt(2.0)) * _np_erfinv_f32(u)).reshape(shape)


def _noise_consts(B, L):
    o0, o1 = _np_threefry2x32(
        np.uint32(0), np.uint32(42),
        np.zeros(2, np.uint32), np.arange(2, dtype=np.uint32))
    base = _np_normal(o0[0], o1[0], (B, L, 3)) * np.float32(_NOISE_STD)
    lcn = _np_normal(o0[1], o1[1], (B, 8, 3)) * np.float32(_NOISE_STD)
    base_planar = np.ascontiguousarray(
        base.transpose(0, 2, 1)).reshape(B, 3 * L)
    lcn_pad = np.zeros((B, 32), np.float32)
    lcn_pad[:, :24] = lcn.reshape(B, 24)
    return base_planar, lcn_pad


_BASE_PLANAR, _LCN_PAD = _noise_consts(_B, _L)


def kernel(x_start, noise_step, non_atom_mask, is_stable_periodic):
    B, L = x_start.shape[0], x_start.shape[1]
    base_planar, lcn_pad = _BASE_PLANAR, _LCN_PAD

    xn, nz, sig, wt = _sc_call(
        x_start.reshape(B, 3 * L),
        base_planar,
        noise_step,
        non_atom_mask.astype(jnp.float32),
        lcn_pad,
        is_stable_periodic.astype(jnp.float32),
    )
    return (xn.reshape(B, L, 3), nz.reshape(B, L, 3),
            sig[..., None], wt[..., None])


# speedup vs baseline: 1.0128x; 1.0128x over previous
import functools

import numpy as np

import jax
import jax.numpy as jnp
from jax import lax
from jax.experimental import pallas as pl
from jax.experimental.pallas import tpu as pltpu
from jax.experimental.pallas import tpu_sc as plsc

_NOISE_STD = 1.0
_P_MEAN = -1.2
_P_STD = 1.5
_SIGMA_DATA = 0.5

_B = 64
_L = 1024
_NC = 2
_NS = 16
_ROWS_PER_W = _B // (_NC * _NS)
_NT = _L // 16


def _sc_body(x_hbm, basep_hbm, ns_hbm, mask_hbm, lcn_hbm, stab_hbm,
             xn_hbm, nz_hbm, sig_hbm, wt_hbm,
             v_x0, v_x1, v_bp0, v_bp1, v_nz0, v_nz1, v_xn0, v_xn1,
             v_mask0, v_mask1, v_ns0, v_ns1, v_sig0, v_sig1,
             v_wt0, v_wt1, v_lcn0, v_lcn1, v_stab, v_mw0, v_mw1,
             sem_in0, sem_in1, sem_out):
    wid = lax.axis_index("s") * _NC + lax.axis_index("c")
    pltpu.sync_copy(stab_hbm, v_stab)

    iota = lax.iota(jnp.int32, 16)
    i3 = iota * 3

    bufs = (
        (v_x0, v_bp0, v_nz0, v_xn0, v_mask0, v_ns0, v_sig0, v_wt0, v_lcn0, v_mw0),
        (v_x1, v_bp1, v_nz1, v_xn1, v_mask1, v_ns1, v_sig1, v_wt1, v_lcn1, v_mw1),
    )
    sems_in = (sem_in0, sem_in1)
    in_handles = []
    for rr in range(_ROWS_PER_W):
        r = wid * _ROWS_PER_W + rr
        sem = sems_in[rr]
        b_x, b_bp, b_nz, b_xn, b_mask, b_ns, b_sig, b_wt, b_lcn, b_maskw = bufs[rr]
        in_handles.append([
            pltpu.async_copy(x_hbm.at[r], b_x, sem),
            pltpu.async_copy(basep_hbm.at[r], b_bp, sem),
            pltpu.async_copy(ns_hbm.at[r], b_ns, sem),
            pltpu.async_copy(mask_hbm.at[r], b_maskw, sem),
            pltpu.async_copy(lcn_hbm.at[r], b_lcn, sem),
        ])

    out_handles = []
    for rr in range(_ROWS_PER_W):
        r = wid * _ROWS_PER_W + rr
        for h in in_handles[rr]:
            h.wait()
        b_x, b_bp, b_nz, b_xn, b_mask, b_ns, b_sig, b_wt, b_lcn, b_maskw = bufs[rr]
        stab_v = plsc.load_gather(v_stab, [jnp.full((16,), r, jnp.int32)])

        @plsc.parallel_loop(0, _L // 64, unroll=2)
        def expand_mask(g, _i3=i3):
            w = b_maskw[pl.ds(g * 16, 16)]
            base_i = iota * 4 + g * 64
            for sbyte in range(4):
                keep = 1.0 - ((w >> (8 * sbyte)) & 1).astype(jnp.float32)
                plsc.store_scatter(b_mask, [base_i + sbyte], keep)

        z = jnp.zeros((16,), jnp.float32)

        @plsc.parallel_loop(0, _NT, unroll=4, carry=(z, z, z, z))
        def pass_a(t, carry):
            accn, ax, ay, az = carry
            o = t * 16
            k = b_mask[pl.ds(o, 16)]
            gx = b_bp[pl.ds(o, 16)]
            gy = b_bp[pl.ds(_L + o, 16)]
            gz = b_bp[pl.ds(2 * _L + o, 16)]
            return (accn + k, ax + k * gx, ay + k * gy, az + k * gz)

        accn, ax, ay, az = pass_a
        n_at = jnp.sum(accn)
        inv = (1.0 - stab_v) / jnp.maximum(jnp.broadcast_to(n_at, (16,)), 1.0)
        cx = jnp.broadcast_to(jnp.sum(ax), (16,)) * inv
        cy = jnp.broadcast_to(jnp.sum(ay), (16,)) * inv
        cz = jnp.broadcast_to(jnp.sum(az), (16,)) * inv

        @plsc.parallel_loop(0, _NT, unroll=4)
        def pass_b(t):
            o = t * 16
            k = b_mask[pl.ds(o, 16)]
            s = _SIGMA_DATA * jnp.exp(b_ns[pl.ds(o, 16)] * _P_STD + _P_MEAN)
            b_sig[pl.ds(o, 16)] = s
            b_wt[pl.ds(o, 16)] = 4.0 + 1.0 / (s * s)
            off = t * 48
            for c, cc in enumerate((cx, cy, cz)):
                idx = i3 + (off + c)
                nf = k * (b_bp[pl.ds(c * _L + o, 16)] - cc)
                plsc.store_scatter(b_nz, [idx], nf)
                xg = plsc.load_gather(b_x, [idx])
                plsc.store_scatter(b_xn, [idx], xg + s * nf)

        @pl.when(stab_v[0] > 0.5)
        def _():
            na_i = n_at.astype(jnp.int32)
            for chunk in range(2):
                i = iota + chunk * 16
                j = i // 3
                c = i - j * 3
                b2 = ((j >> 2) & 1).astype(jnp.float32) - 0.5
                b1 = ((j >> 1) & 1).astype(jnp.float32) - 0.5
                b0 = (j & 1).astype(jnp.float32) - 0.5
                g0 = plsc.load_gather(b_lcn, [c])
                d4 = plsc.load_gather(b_lcn, [c + 12]) - g0
                d2 = plsc.load_gather(b_lcn, [c + 6]) - g0
                d1 = plsc.load_gather(b_lcn, [c + 3]) - g0
                corner = b2 * d4 + b1 * d2 + b0 * d1
                lidx = jnp.minimum(na_i + j, _L - 1)
                tidx = lidx * 3 + c
                valid = i < 24
                plsc.store_scatter(b_nz, [tidx], corner, mask=valid)
                sg = plsc.load_gather(b_sig, [lidx])
                xg = plsc.load_gather(b_x, [tidx])
                plsc.store_scatter(b_xn, [tidx], xg + sg * corner, mask=valid)

        out_handles.extend([
            pltpu.async_copy(b_nz, nz_hbm.at[r], sem_out),
            pltpu.async_copy(b_xn, xn_hbm.at[r], sem_out),
            pltpu.async_copy(b_sig, sig_hbm.at[r], sem_out),
            pltpu.async_copy(b_wt, wt_hbm.at[r], sem_out),
        ])

    for h in out_handles:
        h.wait()


_f32 = jnp.float32
_R = _ROWS_PER_W
_sc_call = pl.kernel(
    _sc_body,
    out_type=(
        jax.ShapeDtypeStruct((_B, 3 * _L), _f32),
        jax.ShapeDtypeStruct((_B, 3 * _L), _f32),
        jax.ShapeDtypeStruct((_B, _L), _f32),
        jax.ShapeDtypeStruct((_B, _L), _f32),
    ),
    mesh=plsc.VectorSubcoreMesh(
        core_axis_name="c", subcore_axis_name="s",
        num_cores=_NC, num_subcores=_NS),
    compiler_params=pltpu.CompilerParams(
        needs_layout_passes=False, disable_bounds_checks=True),
    scratch_types=(
        [pltpu.VMEM((3 * _L,), _f32)] * 8
        + [pltpu.VMEM((_L,), _f32)] * 8
        + [pltpu.VMEM((32,), _f32)] * 2
        + [pltpu.VMEM((_B,), _f32)]
        + [pltpu.VMEM((_L // 4,), jnp.int32)] * 2
        + [pltpu.SemaphoreType.DMA] * 3
    ),
)


def _np_threefry2x32(k0, k1, x0, x1):
    k0 = np.uint32(k0); k1 = np.uint32(k1)
    ks = (k0, k1, np.uint32(k0 ^ k1 ^ np.uint32(0x1BD11BDA)))
    x0 = (x0 + ks[0]).astype(np.uint32)
    x1 = (x1 + ks[1]).astype(np.uint32)
    rots = ([13, 15, 26, 6], [17, 29, 16, 24])
    for i in range(5):
        for r in rots[i % 2]:
            x0 = (x0 + x1).astype(np.uint32)
            x1 = ((x1 << np.uint32(r)) | (x1 >> np.uint32(32 - r))).astype(np.uint32)
            x1 = (x1 ^ x0).astype(np.uint32)
        x0 = (x0 + ks[(i + 1) % 3]).astype(np.uint32)
        x1 = (x1 + ks[(i + 2) % 3] + np.uint32(i + 1)).astype(np.uint32)
    return x0, x1


def _np_random_bits(k0, k1, n):
    o0, o1 = _np_threefry2x32(
        k0, k1, np.zeros(n, np.uint32), np.arange(n, dtype=np.uint32))
    return (o0 ^ o1).astype(np.uint32)


def _np_erfinv_f32(x):
    x = x.astype(np.float32)
    w = (-np.log((np.float32(1.0) - x) * (np.float32(1.0) + x))).astype(np.float32)
    wa = (w - np.float32(2.5)).astype(np.float32)
    p = np.float32(2.81022636e-08)
    for c in (3.43273939e-07, -3.5233877e-06, -4.39150654e-06, 0.00021858087,
              -0.00125372503, -0.00417768164, 0.246640727, 1.50140941):
        p = (np.float32(c) + p * wa).astype(np.float32)
    pa = p
    wb = (np.sqrt(w) - np.float32(3.0)).astype(np.float32)
    p = np.float32(-0.000200214257)
    for c in (0.000100950558, 0.00134934322, -0.00367342844, 0.00573950773,
              -0.0076224613, 0.00943887047, 1.00167406, 2.83297682):
        p = (np.float32(c) + p * wb).astype(np.float32)
    pb = p
    return (np.where(w < np.float32(5.0), pa, pb) * x).astype(np.float32)


def _np_normal(k0, k1, shape):
    n = int(np.prod(shape))
    bits = _np_random_bits(k0, k1, n)
    fb = ((bits >> np.uint32(9)) | np.uint32(0x3F800000)).view(np.float32)
    floats = (fb - np.float32(1.0)).astype(np.float32)
    lo = np.float32(np.nextafter(np.float32(-1.0), np.float32(0.0)))
    u = np.maximum(lo, (floats * (np.float32(1.0) - lo) + lo).astype(np.float32))
    return (np.float32(np.sqrt(2.0)) * _np_erfinv_f32(u)).reshape(shape)


def _noise_consts(B, L):
    o0, o1 = _np_threefry2x32(
        np.uint32(0), np.uint32(42),
        np.zeros(2, np.uint32), np.arange(2, dtype=np.uint32))
    base = _np_normal(o0[0], o1[0], (B, L, 3)) * np.float32(_NOISE_STD)
    lcn = _np_normal(o0[1], o1[1], (B, 8, 3)) * np.float32(_NOISE_STD)
    base_planar = np.ascontiguousarray(
        base.transpose(0, 2, 1)).reshape(B, 3 * L)
    lcn_pad = np.zeros((B, 32), np.float32)
    lcn_pad[:, :24] = lcn.reshape(B, 24)
    return base_planar, lcn_pad


_BASE_PLANAR, _LCN_PAD = _noise_consts(_B, _L)


def kernel(x_start, noise_step, non_atom_mask, is_stable_periodic):
    B, L = x_start.shape[0], x_start.shape[1]
    base_planar, lcn_pad = _BASE_PLANAR, _LCN_PAD

    xn, nz, sig, wt = _sc_call(
        x_start.reshape(B, 3 * L),
        base_planar,
        noise_step,
        non_atom_mask.view(jnp.uint8).view(jnp.int32),
        lcn_pad,
        is_stable_periodic.astype(jnp.float32),
    )
    return (xn.reshape(B, L, 3), nz.reshape(B, L, 3),
            sig[..., None], wt[..., None])
